# baseline (speedup 1.0000x reference)
import functools

import jax
import jax.numpy as jnp
from jax.experimental import pallas as pl

_TOPK = 128


def _body(vr_ref, vf_ref, st_ref, dv_ref, ds_ref, *, topk):
    min32 = jnp.int32(-2147483648)
    m31 = jnp.int32(0x7FFFFFFF)
    vr = vr_ref[0]
    vf = vf_ref[0]
    sT = jax.lax.dot_general(
        vf, vr, (((1,), (1,)), ((), ())),
        preferred_element_type=jnp.float32)
    r = sT.shape[1]

    def to_float(kint):
        fb = kint ^ ((kint >> 31) & m31)
        return jax.lax.bitcast_convert_type(fb, jnp.float32)

    def step(t, p):
        j = 31 - t
        trial = p | (jnp.int32(1) << j)
        ft = to_float(trial ^ min32)
        cnt = jnp.sum((sT >= ft).astype(jnp.int32), axis=0, keepdims=True)
        return jnp.where(cnt >= topk, trial, p)

    p = jax.lax.fori_loop(0, 32, step, jnp.zeros((1, r), jnp.int32))
    tf = to_float(p ^ min32)

    edges = jnp.where(sT >= tf, sT / (1.0 + jnp.abs(sT)), 0.0)
    stc = jnp.swapaxes(st_ref[0], 0, 1)
    ds_ref[0, 0, 0, :] = jnp.sum(edges * stc, axis=0)
    dv_ref[0] = jax.lax.dot_general(
        edges, vf, (((0,), (0,)), ((), ())),
        preferred_element_type=jnp.float32)


@jax.jit
def kernel(val, state):
    b, n, d = val.shape
    r = min(256, n)
    nb = n // r
    topk = min(_TOPK, n)

    grid = (b, nb)
    dv, ds = pl.pallas_call(
        functools.partial(_body, topk=topk),
        grid=grid,
        in_specs=[
            pl.BlockSpec((1, r, d), lambda bi, i: (bi, i, 0)),
            pl.BlockSpec((1, n, d), lambda bi, i: (bi, 0, 0)),
            pl.BlockSpec((1, 1, n), lambda bi, i: (bi, 0, 0)),
        ],
        out_specs=[
            pl.BlockSpec((1, r, d), lambda bi, i: (bi, i, 0)),
            pl.BlockSpec((1, 1, 1, r), lambda bi, i: (bi, i, 0, 0)),
        ],
        out_shape=[
            jax.ShapeDtypeStruct((b, n, d), jnp.float32),
            jax.ShapeDtypeStruct((b, nb, 1, r), jnp.float32),
        ],
    )(val, val, state.reshape(b, 1, n))
    return ds.reshape(b, n), dv


# speedup vs baseline: 1.1509x; 1.1509x over previous
import functools

import jax
import jax.numpy as jnp
from jax.experimental import pallas as pl

_TOPK = 128


def _body(vr_ref, vf_ref, st_ref, dv_ref, ds_ref, *, topk):
    min32 = jnp.int32(-2147483648)
    vr = vr_ref[0]
    vf = vf_ref[0]
    s = jax.lax.dot_general(
        vr, vf, (((1,), (1,)), ((), ())),
        preferred_element_type=jnp.float32)

    bits = jax.lax.bitcast_convert_type(s, jnp.int32)
    key = bits ^ ((bits >> 31) & jnp.int32(0x7FFFFFFF))

    r = s.shape[0]

    def step(t, p):
        j = 31 - t
        trial = p | (jnp.int32(1) << j)
        thresh = trial ^ min32
        cnt = jnp.sum((key >= thresh).astype(jnp.int32), axis=1,
                      keepdims=True)
        return jnp.where(cnt >= topk, trial, p)

    p = jax.lax.fori_loop(0, 32, step, jnp.zeros((r, 1), jnp.int32))
    mask = key >= (p ^ min32)

    edges = jnp.where(mask, s / (1.0 + jnp.abs(s)), 0.0)
    ds_ref[0, 0, 0, :] = jnp.sum(edges * st_ref[0, 0, :][None, :], axis=1)
    dv_ref[0] = jax.lax.dot_general(
        edges, vf, (((1,), (0,)), ((), ())),
        preferred_element_type=jnp.float32)


@jax.jit
def kernel(val, state):
    b, n, d = val.shape
    r = min(256, n)
    nb = n // r
    topk = min(_TOPK, n)

    grid = (b, nb)
    dv, ds = pl.pallas_call(
        functools.partial(_body, topk=topk),
        grid=grid,
        in_specs=[
            pl.BlockSpec((1, r, d), lambda bi, i: (bi, i, 0)),
            pl.BlockSpec((1, n, d), lambda bi, i: (bi, 0, 0)),
            pl.BlockSpec((1, 1, n), lambda bi, i: (bi, 0, 0)),
        ],
        out_specs=[
            pl.BlockSpec((1, r, d), lambda bi, i: (bi, i, 0)),
            pl.BlockSpec((1, 1, 1, r), lambda bi, i: (bi, i, 0, 0)),
        ],
        out_shape=[
            jax.ShapeDtypeStruct((b, n, d), jnp.float32),
            jax.ShapeDtypeStruct((b, nb, 1, r), jnp.float32),
        ],
    )(val, val, state.reshape(b, 1, n))
    return ds.reshape(b, n), dv
